# baseline (speedup 1.0000x reference)
import functools

import jax
import jax.numpy as jnp
from jax import lax
from jax.experimental import pallas as pl
from jax.experimental.pallas import tpu as pltpu
from jax.experimental.pallas import tpu_sc as plsc

EMBED_DIM = 64
CHUNK = 128


@jax.jit
def _embed(idx3, weight):
    info = plsc.get_sparse_core_info()
    nw = info.num_cores * info.num_subcores
    n_chunks = idx3.shape[1]
    per_w = n_chunks * CHUNK
    n = nw * per_w

    mesh = plsc.VectorSubcoreMesh(core_axis_name="c", subcore_axis_name="s")

    @functools.partial(
        pl.kernel,
        mesh=mesh,
        compiler_params=pltpu.CompilerParams(use_tc_tiling_on_sc=False),
        out_type=jax.ShapeDtypeStruct((n, EMBED_DIM), jnp.float32),
        scratch_types=[
            pltpu.VMEM((n_chunks, CHUNK), jnp.int32),
            pltpu.VMEM((2, CHUNK, EMBED_DIM), jnp.float32),
            pltpu.SemaphoreType.DMA,
            pltpu.SemaphoreType.DMA,
        ],
    )
    def emb(idx_hbm, table_hbm, out_hbm, idx_v, rows_v, g0, g1):
        wid = lax.axis_index("c") * info.num_subcores + lax.axis_index("s")
        base = wid * per_w
        pltpu.sync_copy(idx_hbm.at[wid], idx_v)

        gsem = (g0, g1)

        def gath(j):
            return pltpu.make_async_copy(
                table_hbm.at[idx_v.at[j]],
                rows_v.at[j % 2],
                gsem[j % 2],
            )

        gath(0).start()
        for j in range(n_chunks):
            if j + 1 < n_chunks:
                gath(j + 1).start()
            gath(j).wait()
            pltpu.sync_copy(
                rows_v.at[j % 2],
                out_hbm.at[pl.ds(base + j * CHUNK, CHUNK)],
            )

    return emb(idx3, weight)


def kernel(input, weight):
    b, h = input.shape
    n = b * h
    info = plsc.get_sparse_core_info()
    nw = info.num_cores * info.num_subcores
    idx3 = input.reshape(nw, n // (nw * CHUNK), CHUNK).astype(jnp.int32)
    out = _embed(idx3, weight)
    return out.reshape(b, h, weight.shape[1])
